# Initial kernel scaffold; baseline (speedup 1.0000x reference)
import jax
import jax.numpy as jnp
from jax.experimental import pallas as pl


def kernel(pxo, params):
    raise NotImplementedError("write your pallas kernel here")


# speedup vs baseline: 27.2712x; 27.2712x over previous
import functools
import numpy as np
import jax
import jax.numpy as jnp
from jax import lax
from jax.experimental import pallas as pl
from jax.experimental.pallas import tpu as pltpu
from jax.experimental.pallas import tpu_sc as plsc

_SHARE = 8
_NS = [8, 16, 16, 16, 16]
_PLANES = [32, 64, 128, 256, 512]
_NPTS = [4096, 1024, 256, 64, 16]
_GRID = 16
_INTERPRET = False
_NW = 32
_SCCHUNK = 128
_SCBUF = 4

_MB_KEYS = ('W1', 'Wp', 'Wv', 'Wm1', 'Wm2', 'We', 'Wv2', 'W3')


def _knn_np(pq, pr, k):
    d = np.sum((pq[:, None, :].astype(np.float64) - pr[None, :, :].astype(np.float64)) ** 2, axis=-1)
    return np.argsort(d, axis=-1, kind='stable')[:, :k].astype(np.int32)


def _build_consts():
    m = _GRID
    g = np.linspace(0, m - 1, m)
    X, Y, Z = np.meshgrid(g, g, g)
    p0 = np.concatenate([X.reshape(-1, 1), Y.reshape(-1, 1), Z.reshape(-1, 1)], 1).astype(np.float32)
    P = [p0]
    for _ in range(4):
        P.append(P[-1][np.arange(P[-1].shape[0] // 4) * 4])

    C = {'p0T': p0.T.copy()}
    for l in range(1, 6):
        p, ns = P[l - 1], _NS[l - 1]
        N = p.shape[0]
        idx = _knn_np(p, p, ns)
        rel = (p[idx] - p[:, None, :]).astype(np.float32)
        idxT = idx.T.reshape(ns * N)
        C[f'in{l}_idxT'] = idxT.reshape(1, -1).copy()
        C[f'in{l}_lanes'] = idx.T.copy()
        C[f'in{l}_relT'] = rel.transpose(1, 0, 2).reshape(ns * N, 3).T.copy()
        if l <= 2:
            C[f'in{l}_sc'] = idxT.reshape(-1, _SCBUF, _SCCHUNK).copy()
    for l in range(2, 6):
        pq, pr, ns = P[l - 1], P[l - 2], _NS[l - 1]
        idx = _knn_np(pq, pr, ns)
        rel = (pr[idx] - pq[:, None, :]).astype(np.float32)
        idxT = idx.T.reshape(-1)
        C[f'td{l}_idxT'] = idxT.reshape(1, -1).copy()
        C[f'td{l}_relT'] = rel.transpose(1, 0, 2).reshape(-1, 3).T.copy()
        if l == 2:
            C[f'td{l}_sc'] = idxT.reshape(-1, _SCBUF, _SCCHUNK).copy()
    for l in range(1, 5):
        p1, p2 = P[l - 1], P[l]
        idx = _knn_np(p1, p2, 3)
        d = np.sum((p1[:, None, :] - p2[idx]) ** 2, axis=-1).astype(np.float32)
        w = (np.float32(1.0) / (d + np.float32(1e-8))).astype(np.float32)
        w = w / np.sum(w, axis=1, keepdims=True, dtype=np.float32)
        C[f'tu{l}_idxT'] = idx.T.reshape(1, -1).copy()
        C[f'tu{l}_wT'] = w.T.reshape(1, -1).astype(np.float32).copy()
    return C


_CONSTS = _build_consts()


def _sc_gather(table, idx_sc):
    nrow, nchb, chk = idx_sc.shape
    R = nrow // _NW
    D = table.shape[1]
    mesh = plsc.VectorSubcoreMesh(core_axis_name="c", subcore_axis_name="s")

    @functools.partial(
        pl.kernel, mesh=mesh,
        out_type=jax.ShapeDtypeStruct((nrow, nchb, chk, D), jnp.float32),
        scratch_types=[
            pltpu.VMEM((nchb, chk), jnp.int32),
            pltpu.VMEM((nchb, chk, D), jnp.float32),
            pltpu.SemaphoreType.DMA,
        ],
    )
    def gk(table_hbm, idx_hbm, out_hbm, idx_v, rows_v, sem):
        wid = lax.axis_index("s") * 2 + lax.axis_index("c")
        for r in range(R):
            row = wid * R + r
            pltpu.sync_copy(idx_hbm.at[row], idx_v)
            descs = []
            for k in range(nchb):
                descs.append(pltpu.async_copy(table_hbm.at[idx_v.at[k]], rows_v.at[k], sem))
            for dsc in descs:
                dsc.wait()
            pltpu.sync_copy(rows_v, out_hbm.at[row])

    out = gk(table, idx_sc)
    return out.reshape(nrow * nchb * chk, D)


def _expand_mat(C8, Cc):
    r = lax.broadcasted_iota(jnp.int32, (C8, Cc), 0)
    c = lax.broadcasted_iota(jnp.int32, (C8, Cc), 1)
    return (r == (c // _SHARE)).astype(jnp.float32)


def _dot(a, b, hi=False):
    return jnp.dot(a, b, preferred_element_type=jnp.float32,
                   precision=lax.Precision.HIGHEST if hi else None)


def _dotT(a, b, hi=False):
    return lax.dot_general(a, b, (((0,), (0,)), ((), ())),
                           preferred_element_type=jnp.float32,
                           precision=lax.Precision.HIGHEST if hi else None)


def _onehotT(idx_row, Nsrc):
    iota = lax.broadcasted_iota(jnp.int32, (Nsrc, idx_row.shape[1]), 0)
    return (idx_row == iota).astype(jnp.float32)


def _mixer_head(x, d, b, pad):
    xp = jax.nn.relu(_dot(x, d[f'{b}.W1'][...]))
    xv = _dot(xp, d[f'{b}.Wv'][...])
    parts = [xp, xv]
    if pad:
        parts.append(jnp.zeros((x.shape[0], pad), jnp.float32))
    return jnp.concatenate(parts, axis=1)


def _intra_tail(hs, vs, ns, R):
    m = hs[0]
    for j in range(1, ns):
        m = jnp.maximum(m, hs[j])
    es = [jnp.exp(h - m) for h in hs]
    s = es[0]
    for j in range(1, ns):
        s = s + es[j]
    acc = None
    for j in range(ns):
        t = _dot(es[j] / s, R, hi=True) * vs[j]
        acc = t if acc is None else acc + t
    return acc


def _intra_from_G(G_ref, l, b, d, xi_ref, nch):
    N, Cc, ns = _NPTS[l - 1], _PLANES[l - 1], _NS[l - 1]
    C8 = Cc // _SHARE
    R = _expand_mat(C8, Cc)
    Wp = d[f'{b}.Wp'][...]
    Wm1 = d[f'{b}.Wm1'][...]
    Wm2 = d[f'{b}.Wm2'][...]
    relT = d[f'in{l}_relT']
    Bp = N // nch

    def chunk(i, _):
        base = i * Bp
        hs, vs = [], []
        for j in range(ns):
            Gj = G_ref[pl.ds(j * N + base, Bp), :]
            gx, gv = Gj[:, :Cc], Gj[:, Cc:2 * Cc]
            pe = _dotT(relT[:, pl.ds(j * N + base, Bp)], Wp)
            wj = _dot(jax.nn.relu(_dot(gx + pe, Wm1)), Wm2)
            hs.append(wj)
            vs.append(gv + pe)
        xi_ref[pl.ds(base, Bp), :] = _intra_tail(hs, vs, ns, R)
        return 0

    if nch == 1:
        chunk(0, 0)
    else:
        lax.fori_loop(0, nch, chunk, 0)


def _intra_onehot(x_in, l, b, d, xi_ref):
    N, Cc, ns = _NPTS[l - 1], _PLANES[l - 1], _NS[l - 1]
    C8 = Cc // _SHARE
    R = _expand_mat(C8, Cc)
    x = jax.nn.relu(_dot(x_in, d[f'{b}.W1'][...]))
    xv = _dot(x, d[f'{b}.Wv'][...])
    Wp = d[f'{b}.Wp'][...]
    Wm1 = d[f'{b}.Wm1'][...]
    Wm2 = d[f'{b}.Wm2'][...]
    idxT = d[f'in{l}_idxT']
    relT = d[f'in{l}_relT']
    hs, vs = [], []
    for j in range(ns):
        oh = _onehotT(idxT[:, pl.ds(j * N, N)], N)
        gx = _dotT(oh, x, hi=True)
        gv = _dotT(oh, xv, hi=True)
        pe = _dotT(relT[:, pl.ds(j * N, N)], Wp)
        wj = _dot(jax.nn.relu(_dot(gx + pe, Wm1)), Wm2)
        hs.append(wj)
        vs.append(gv + pe)
    xi_ref[...] = _intra_tail(hs, vs, ns, R)


def _inter_finish(iden, l, b, d, xi_ref, nd_ref, nch):
    N, Cc, ns = _NPTS[l - 1], _PLANES[l - 1], _NS[l - 1]
    C8 = Cc // _SHARE
    R = _expand_mat(C8, Cc)
    xi = xi_ref[...]
    e = _dot(xi, d[f'{b}.We'][...])
    v2 = _dot(xi, d[f'{b}.Wv2'][...])
    mx = jnp.max(e, axis=(0, 1), keepdims=True)
    expe = jnp.exp(e - mx)
    contrib = v2 * _dot(expe, R, hi=True)
    tbl = jnp.concatenate([contrib, expe], axis=1)
    lanes = d[f'in{l}_lanes']
    Bd = N // nch

    def chunk(i, _):
        base = i * Bd
        dio = lax.broadcasted_iota(jnp.int32, (Bd, N), 0) + base
        acc = None
        for j in range(ns):
            msk = (lanes[pl.ds(j, 1), :] == dio).astype(jnp.float32)
            acc = msk if acc is None else acc + msk
        nd_ref[pl.ds(base, Bd), :] = _dot(acc, tbl, hi=True)
        return 0

    if nch == 1:
        chunk(0, 0)
    else:
        lax.fori_loop(0, nch, chunk, 0)

    nd = nd_ref[...]
    den = _dot(nd[:, Cc:] + 1e-8, R, hi=True)
    x2 = jax.nn.relu(xi + nd[:, :Cc] / den)
    return jax.nn.relu(_dot(x2, d[f'{b}.W3'][...]) + iden)


def _td_from_G(G_ref, l, d, Cin):
    Nq, ns = _NPTS[l - 1], _NS[l - 1]
    W = d[f'td{l}.W']
    Wtop, Wbot = W[0:3, :], W[3:, :]
    relT = d[f'td{l}_relT']
    m = None
    for j in range(ns):
        gj = G_ref[pl.ds(j * Nq, Nq), :][:, :Cin]
        pe = _dotT(relT[:, pl.ds(j * Nq, Nq)], Wtop)
        fj = jax.nn.relu(pe + _dot(gj, Wbot))
        m = fj if m is None else jnp.maximum(m, fj)
    return m


def _td_onehot(x, l, d):
    Nq, ns = _NPTS[l - 1], _NS[l - 1]
    Nsrc = _NPTS[l - 2]
    W = d[f'td{l}.W']
    Wtop, Wbot = W[0:3, :], W[3:, :]
    idxT = d[f'td{l}_idxT']
    relT = d[f'td{l}_relT']
    m = None
    for j in range(ns):
        oh = _onehotT(idxT[:, pl.ds(j * Nq, Nq)], Nsrc)
        pe = _dotT(relT[:, pl.ds(j * Nq, Nq)], Wtop)
        fj = jax.nn.relu(pe + _dot(_dotT(oh, x, hi=True), Wbot))
        m = fj if m is None else jnp.maximum(m, fj)
    return m


def _tu_onehot(y2, l, d, nch, interp_ref):
    N1, N2 = _NPTS[l - 1], _NPTS[l]
    idxT = d[f'tu{l}_idxT']
    wT = d[f'tu{l}_wT']
    Bq = N1 // nch

    def chunk(i, _):
        base = i * Bq
        acc = None
        for j in range(3):
            ohw = _onehotT(idxT[:, pl.ds(j * N1 + base, Bq)], N2) \
                * wT[:, pl.ds(j * N1 + base, Bq)]
            t = _dotT(ohw, y2, hi=True)
            acc = t if acc is None else acc + t
        interp_ref[pl.ds(base, Bq), :] = acc
        return 0

    if nch == 1:
        chunk(0, 0)
    else:
        lax.fori_loop(0, nch, chunk, 0)
    return interp_ref[...]


def _make_stage(names, out_shapes, scratch, body):
    def kernel_fn(*refs):
        d = dict(zip(names, refs[:len(names)]))
        outs = refs[len(names):len(names) + len(out_shapes)]
        scr = refs[len(names) + len(out_shapes):]
        body(d, outs, scr)

    def call(arrs):
        return pl.pallas_call(
            kernel_fn,
            out_shape=[jax.ShapeDtypeStruct(s, jnp.float32) for s in out_shapes],
            scratch_shapes=[pltpu.VMEM(s, jnp.float32) for s in scratch],
            interpret=_INTERPRET,
        )(*[arrs[n] for n in names])
    return call


def kernel(pxo, params):
    arrs = {k: jnp.asarray(v) for k, v in _CONSTS.items()}
    arrs['v0'] = pxo.reshape(1, -1).astype(jnp.float32)
    for b in ['enc1_mb', 'enc2_mb', 'enc3_mb', 'enc4_mb', 'enc5_mb',
              'dec5_mb', 'dec4_mb', 'dec3_mb', 'dec2_mb', 'dec1_mb']:
        for k in _MB_KEYS:
            arrs[f'{b}.{k}'] = params[b][k]
    arrs['td1.W'] = params['enc1_td']['W']
    for l in range(2, 6):
        arrs[f'td{l}.W'] = params[f'enc{l}_td']['W']
    for l in range(1, 6):
        arrs[f'up{l}.W1'] = params[f'dec{l}_up']['W1']
        arrs[f'up{l}.W2'] = params[f'dec{l}_up']['W2']
    arrs['cls.Wa'] = params['cls']['Wa']
    arrs['cls.ba'] = params['cls']['ba'].reshape(1, -1)
    arrs['cls.Wb'] = params['cls']['Wb']
    arrs['cls.bb'] = params['cls']['bb'].reshape(1, -1)

    def tc1_body(d, outs, scr):
        W = d['td1.W']
        x1 = jax.nn.relu(_dotT(d['p0T'][...], W[0:3, :]) + _dotT(d['v0'][...], W[3:4, :]))
        outs[0][...] = x1
        outs[1][...] = _mixer_head(x1, d, 'enc1_mb', 64)
    tc1 = _make_stage(['p0T', 'v0', 'td1.W', 'enc1_mb.W1', 'enc1_mb.Wv'],
                      [(4096, 32), (4096, 128)], [], tc1_body)
    x1, T1 = tc1(arrs)

    arrs['G1'] = _sc_gather(T1, arrs['in1_sc'])
    arrs['x1'] = x1

    def tc2_body(d, outs, scr):
        _intra_from_G(d['G1'], 1, 'enc1_mb', d, scr[0], 8)
        xb = _inter_finish(d['x1'][...], 1, 'enc1_mb', d, scr[0], scr[1], 32)
        outs[0][...] = jnp.concatenate([xb, jnp.zeros((4096, 96), jnp.float32)], axis=1)
    tc2 = _make_stage(['G1', 'x1', 'in1_relT', 'in1_lanes'] +
                      [f'enc1_mb.{k}' for k in ('Wp', 'Wm1', 'Wm2', 'We', 'Wv2', 'W3')],
                      [(4096, 128)], [(4096, 32), (4096, 36)], tc2_body)
    (x1b,) = tc2(arrs)

    arrs['Gt2'] = _sc_gather(x1b, arrs['td2_sc'])
    arrs['x1b'] = x1b

    def tc3_body(d, outs, scr):
        x2 = _td_from_G(d['Gt2'], 2, d, 32)
        outs[0][...] = x2
        outs[1][...] = _mixer_head(x2, d, 'enc2_mb', 0)
    tc3 = _make_stage(['Gt2', 'td2_relT', 'td2.W', 'enc2_mb.W1', 'enc2_mb.Wv'],
                      [(1024, 64), (1024, 128)], [], tc3_body)
    x2, T1l2 = tc3(arrs)

    arrs['G2'] = _sc_gather(T1l2, arrs['in2_sc'])
    arrs['x2'] = x2

    tc4_names = (['G2', 'x2', 'in2_relT', 'in2_lanes'] +
                 [f'enc2_mb.{k}' for k in ('Wp', 'Wm1', 'Wm2', 'We', 'Wv2', 'W3')] +
                 [f'td{l}_idxT' for l in (3, 4, 5)] + [f'td{l}_relT' for l in (3, 4, 5)] +
                 [f'td{l}.W' for l in (3, 4, 5)] +
                 [f'in{l}_idxT' for l in (3, 4, 5)] + [f'in{l}_relT' for l in (3, 4, 5)] +
                 [f'in{l}_lanes' for l in (3, 4, 5)] +
                 [f'{b}.{k}' for b in ('enc3_mb', 'enc4_mb', 'enc5_mb', 'dec5_mb',
                                       'dec4_mb', 'dec3_mb') for k in _MB_KEYS] +
                 ['up5.W1', 'up5.W2', 'up4.W1', 'up4.W2', 'up3.W1', 'up3.W2',
                  'up2.W1', 'up2.W2'] +
                 [f'tu{l}_idxT' for l in (2, 3, 4)] + [f'tu{l}_wT' for l in (2, 3, 4)] +
                 ['dec2_mb.W1', 'dec2_mb.Wv'])

    def tc4_body(d, outs, scr):
        xi2, nd2 = scr[0], scr[1]
        _intra_from_G(d['G2'], 2, 'enc2_mb', d, xi2, 2)
        x = _inter_finish(d['x2'][...], 2, 'enc2_mb', d, xi2, nd2, 4)
        enc = {2: x}
        si = 2
        for l in (3, 4, 5):
            x = _td_onehot(x, l, d)
            _intra_onehot(x, l, f'enc{l}_mb', d, scr[si])
            x = _inter_finish(x, l, f'enc{l}_mb', d, scr[si], scr[si + 1], 1)
            enc[l] = x
            si += 2
        g = jnp.mean(_dot(x, d['up5.W2'][...]), axis=0, keepdims=True)
        x = jax.nn.relu(_dot(x, d['up5.W1'][...]) + g)
        _intra_onehot(x, 5, 'dec5_mb', d, scr[si])
        x = _inter_finish(x, 5, 'dec5_mb', d, scr[si], scr[si + 1], 1)
        si += 2
        for l in (4, 3, 2):
            y2 = _dot(x, d[f'up{l}.W2'][...])
            interp = _tu_onehot(y2, l, d, 1, scr[si])
            si += 1
            x = jax.nn.relu(_dot(enc[l], d[f'up{l}.W1'][...])) + interp
            if l > 2:
                _intra_onehot(x, l, f'dec{l}_mb', d, scr[si])
                x = _inter_finish(x, l, f'dec{l}_mb', d, scr[si], scr[si + 1], 1)
                si += 2
        outs[0][...] = x
        outs[1][...] = _mixer_head(x, d, 'dec2_mb', 0)

    tc4_scr = [(1024, 64), (1024, 72)]
    for l in (3, 4, 5):
        N, C = _NPTS[l - 1], _PLANES[l - 1]
        tc4_scr += [(N, C), (N, C + C // _SHARE)]
    tc4_scr += [(16, 512), (16, 576)]
    tc4_scr += [(64, 256)]
    tc4_scr += [(64, 256), (64, 288)]
    tc4_scr += [(256, 128)]
    tc4_scr += [(256, 128), (256, 144)]
    tc4_scr += [(1024, 64)]
    tc4 = _make_stage(tc4_names, [(1024, 64), (1024, 128)], tc4_scr, tc4_body)
    xd2_in, T1d2 = tc4(arrs)

    arrs['Gd2'] = _sc_gather(T1d2, arrs['in2_sc'])
    arrs['xd2_in'] = xd2_in

    def tc5_body(d, outs, scr):
        _intra_from_G(d['Gd2'], 2, 'dec2_mb', d, scr[0], 2)
        x2d = _inter_finish(d['xd2_in'][...], 2, 'dec2_mb', d, scr[0], scr[1], 4)
        y2 = _dot(x2d, d['up1.W2'][...])
        interp = _tu_onehot(y2, 1, d, 4, scr[2])
        xd1 = jax.nn.relu(_dot(d['x1b'][...][:, :32], d['up1.W1'][...])) + interp
        outs[0][...] = xd1
        outs[1][...] = _mixer_head(xd1, d, 'dec1_mb', 64)
    tc5 = _make_stage(['Gd2', 'xd2_in', 'x1b', 'in2_relT', 'in2_lanes',
                       'tu1_idxT', 'tu1_wT', 'up1.W1', 'up1.W2',
                       'dec1_mb.W1', 'dec1_mb.Wv'] +
                      [f'dec2_mb.{k}' for k in ('Wp', 'Wm1', 'Wm2', 'We', 'Wv2', 'W3')],
                      [(4096, 32), (4096, 128)],
                      [(1024, 64), (1024, 72), (4096, 32)], tc5_body)
    xd1_in, T1d1 = tc5(arrs)

    arrs['Gd1'] = _sc_gather(T1d1, arrs['in1_sc'])
    arrs['xd1_in'] = xd1_in

    def tc6_body(d, outs, scr):
        _intra_from_G(d['Gd1'], 1, 'dec1_mb', d, scr[0], 8)
        x = _inter_finish(d['xd1_in'][...], 1, 'dec1_mb', d, scr[0], scr[1], 32)
        h = jax.nn.relu(_dot(x, d['cls.Wa'][...]) + d['cls.ba'][...])
        outs[0][...] = _dot(h, d['cls.Wb'][...]) + d['cls.bb'][...]
    tc6 = _make_stage(['Gd1', 'xd1_in', 'in1_relT', 'in1_lanes',
                       'cls.Wa', 'cls.ba', 'cls.Wb', 'cls.bb'] +
                      [f'dec1_mb.{k}' for k in ('Wp', 'Wm1', 'Wm2', 'We', 'Wv2', 'W3')],
                      [(4096, 2)], [(4096, 32), (4096, 36)], tc6_body)
    (out,) = tc6(arrs)
    return out.reshape(1, 2, _GRID, _GRID, _GRID)
